# baseline (speedup 1.0000x reference)
import jax
import jax.numpy as jnp
from jax import lax
from jax.experimental import pallas as pl

B, L, C_IN, D_MODEL = 16, 4096, 7, 1024
TL = 2048

_OFF_MONTH, _OFF_DAY, _OFF_WEEKDAY, _OFF_HOUR = 0, 13, 45, 52
_T_ROWS = 76


def _embed_block(x_ref, idx_ref, pe_ref, w_ref, out_ref):
    l = pl.program_id(0)
    start = l * TL
    main = x_ref[0, pl.ds(start, TL)]
    row_prev = x_ref[0, pl.ds((start - 1) % L, 1)]
    row_next = x_ref[0, pl.ds((start + TL) % L, 1)]
    shift_m1 = jnp.concatenate([row_prev, main[:-1]], axis=0)
    shift_p1 = jnp.concatenate([main[1:], row_next], axis=0)
    xwin = jnp.concatenate([shift_m1, main, shift_p1], axis=1)

    idx = idx_ref[0]
    iota = lax.broadcasted_iota(jnp.int32, (TL, _T_ROWS), 1)
    oh = ((iota == idx[:, 0:1] + _OFF_MONTH)
          | (iota == idx[:, 1:2] + _OFF_DAY)
          | (iota == idx[:, 2:3] + _OFF_WEEKDAY)
          | (iota == idx[:, 3:4] + _OFF_HOUR)).astype(jnp.float32)

    a = jnp.concatenate([xwin, oh], axis=1)
    mm = jnp.dot(a, w_ref[...], preferred_element_type=jnp.float32)
    out_ref[0] = mm + pe_ref[...]


def kernel(x, x_mark, W_conv, pe, hour_t, weekday_t, day_t, month_t):
    wc = jnp.transpose(W_conv, (2, 1, 0)).reshape(3 * C_IN, D_MODEL)
    wfull = jnp.concatenate([wc, month_t, day_t, weekday_t, hour_t], axis=0)

    nl = L // TL
    grid = (nl, B)
    out = pl.pallas_call(
        _embed_block,
        grid=grid,
        in_specs=[
            pl.BlockSpec((1, L, C_IN), lambda l, b: (b, 0, 0)),
            pl.BlockSpec((1, TL, 4), lambda l, b: (b, l, 0)),
            pl.BlockSpec((TL, D_MODEL), lambda l, b: (l, 0)),
            pl.BlockSpec((3 * C_IN + _T_ROWS, D_MODEL), lambda l, b: (0, 0)),
        ],
        out_specs=pl.BlockSpec((1, TL, D_MODEL), lambda l, b: (b, l, 0)),
        out_shape=jax.ShapeDtypeStruct((B, L, D_MODEL), jnp.float32),
    )(x, x_mark, pe, wfull)
    return out
